# baseline (speedup 1.0000x reference)
import jax
import jax.numpy as jnp
from jax import lax
from jax.experimental import pallas as pl
from jax.experimental.pallas import tpu as pltpu
from jax.experimental.pallas import tpu_sc as plsc

_B = 16384
_XW = 16
_DIMS = (3, 4, 6, 4, 50)
_OUT_W = _XW + sum(_DIMS)

_NC, _NS = 2, 16
_NW = _NC * _NS
_BPW = _B // _NW
_NCHK = _BPW // 16

_TBASE = []
_a = 0
for _v, _d in zip((6, 7, 12, 7, 96), _DIMS):
    _TBASE.append(_a)
    _a += -(-(_v * _d) // 8) * 8
_TPACK = _a

_COL_OFF = []
_o = _XW
for _d in _DIMS:
    _COL_OFF.append(_o)
    _o += _d


def _body(xf_hbm, ycols_hbm, tbl_hbm, out_hbm, tbl_v, x_v, y_v, stage_v):
    wid = lax.axis_index("s") * _NC + lax.axis_index("c")
    base = wid * _BPW

    pltpu.sync_copy(tbl_hbm, tbl_v)
    pltpu.sync_copy(xf_hbm.at[pl.ds(base * _XW, _BPW * _XW)], x_v)
    for t in range(5):
        pltpu.sync_copy(
            ycols_hbm.at[pl.ds(t * _B + base, _BPW)],
            y_v.at[pl.ds(t * _BPW, _BPW)],
        )

    iota = jax.lax.iota(jnp.int32, 16)

    @plsc.parallel_loop(0, _NCHK, unroll=2)
    def chunk(c):
        r0 = c * 16
        rows83 = iota * _OUT_W + r0 * _OUT_W

        for k in range(16):
            xvec = x_v[pl.ds((r0 + k) * _XW, 16)]
            plsc.store_scatter(stage_v, [iota + (r0 + k) * _OUT_W], xvec)

        for t in range(5):
            d = _DIMS[t]
            yt = y_v[pl.ds(t * _BPW + r0, 16)]
            srcb = yt * d + _TBASE[t]
            dstb = rows83 + _COL_OFF[t]
            for j in range(d):
                v = plsc.load_gather(tbl_v, [srcb + j])
                plsc.store_scatter(stage_v, [dstb + j], v)

    pltpu.sync_copy(stage_v, out_hbm.at[pl.ds(base * _OUT_W, _BPW * _OUT_W)])


def kernel(x, y, emb0, emb1, emb2, emb3, emb4):
    ycols = y.T.reshape(5 * _B)
    xf = x.reshape(_B * _XW)
    pieces = []
    for e, b, nb in zip((emb0, emb1, emb2, emb3, emb4),
                        _TBASE, _TBASE[1:] + [_TPACK]):
        r = e.reshape(-1)
        pieces.append(r)
        pad = nb - b - r.shape[0]
        if pad:
            pieces.append(jnp.zeros((pad,), jnp.float32))
    tbl = jnp.concatenate(pieces)

    mesh = plsc.VectorSubcoreMesh(core_axis_name="c", subcore_axis_name="s")
    kern = pl.kernel(
        _body,
        out_type=jax.ShapeDtypeStruct((_B * _OUT_W,), jnp.float32),
        mesh=mesh,
        scratch_types=[
            pltpu.VMEM((_TPACK,), jnp.float32),
            pltpu.VMEM((_BPW * _XW,), jnp.float32),
            pltpu.VMEM((5 * _BPW,), jnp.int32),
            pltpu.VMEM((_BPW * _OUT_W,), jnp.float32),
        ],
        compiler_params=pltpu.CompilerParams(needs_layout_passes=False),
    )
    flat = kern(xf, ycols, tbl)
    return flat.reshape(_B, _OUT_W)


# speedup vs baseline: 1.0887x; 1.0887x over previous
import jax
import jax.numpy as jnp
from jax import lax
from jax.experimental import pallas as pl
from jax.experimental.pallas import tpu as pltpu
from jax.experimental.pallas import tpu_sc as plsc

_B = 16384
_XW = 16
_DIMS = (3, 4, 6, 4, 50)
_OUT_W = _XW + sum(_DIMS)

_NC, _NS = 2, 16
_NW = _NC * _NS
_BPW = _B // _NW
_NCHK = _BPW // 16
_HCHK = _NCHK // 2
_HW = _BPW * _OUT_W // 2

_TBASE = []
_a = 0
for _v, _d in zip((6, 7, 12, 7, 96), _DIMS):
    _TBASE.append(_a)
    _a += -(-(_v * _d) // 8) * 8
_TPACK = _a

_COL_OFF = []
_o = _XW
for _d in _DIMS:
    _COL_OFF.append(_o)
    _o += _d


def _body(xf_hbm, yw_hbm, tbl_hbm, out_hbm, tbl_v, x_v, y_v, stage_v,
          in_sem, st_sem):
    wid = lax.axis_index("s") * _NC + lax.axis_index("c")
    base = wid * _BPW

    c1 = pltpu.make_async_copy(tbl_hbm, tbl_v, in_sem)
    c2 = pltpu.make_async_copy(
        xf_hbm.at[pl.ds(base * _XW, _BPW * _XW)], x_v, in_sem)
    c3 = pltpu.make_async_copy(
        yw_hbm.at[pl.ds(wid * 5 * _BPW, 5 * _BPW)], y_v, in_sem)
    c1.start()
    c2.start()
    c3.start()
    c1.wait()
    c2.wait()
    c3.wait()

    iota = jax.lax.iota(jnp.int32, 16)

    def chunk(c, carry):
        r0 = c * 16
        rows83 = iota * _OUT_W + r0 * _OUT_W

        for k in range(16):
            xvec = x_v[pl.ds((r0 + k) * _XW, 16)]
            plsc.store_scatter(stage_v, [iota + (r0 + k) * _OUT_W], xvec)

        for t in range(5):
            d = _DIMS[t]
            yt = y_v[pl.ds(t * _BPW + r0, 16)]
            srcb = yt * d + _TBASE[t]
            dstb = rows83 + _COL_OFF[t]
            for j in range(d):
                v = plsc.load_gather(tbl_v, [srcb + j])
                plsc.store_scatter(stage_v, [dstb + j], v)
        return carry

    lax.fori_loop(0, _HCHK, chunk, 0)
    s1 = pltpu.make_async_copy(
        stage_v.at[pl.ds(0, _HW)],
        out_hbm.at[pl.ds(base * _OUT_W, _HW)], st_sem)
    s1.start()
    lax.fori_loop(_HCHK, _NCHK, chunk, 0)
    s2 = pltpu.make_async_copy(
        stage_v.at[pl.ds(_HW, _HW)],
        out_hbm.at[pl.ds(base * _OUT_W + _HW, _HW)], st_sem)
    s2.start()
    s1.wait()
    s2.wait()


def kernel(x, y, emb0, emb1, emb2, emb3, emb4):
    yw = y.T.reshape(5, _NW, _BPW).transpose(1, 0, 2).reshape(-1)
    xf = x.reshape(_B * _XW)
    pieces = []
    for e, b, nb in zip((emb0, emb1, emb2, emb3, emb4),
                        _TBASE, _TBASE[1:] + [_TPACK]):
        r = e.reshape(-1)
        pieces.append(r)
        pad = nb - b - r.shape[0]
        if pad:
            pieces.append(jnp.zeros((pad,), jnp.float32))
    tbl = jnp.concatenate(pieces)

    mesh = plsc.VectorSubcoreMesh(core_axis_name="c", subcore_axis_name="s")
    kern = pl.kernel(
        _body,
        out_type=jax.ShapeDtypeStruct((_B * _OUT_W,), jnp.float32),
        mesh=mesh,
        scratch_types=[
            pltpu.VMEM((_TPACK,), jnp.float32),
            pltpu.VMEM((_BPW * _XW,), jnp.float32),
            pltpu.VMEM((5 * _BPW,), jnp.int32),
            pltpu.VMEM((_BPW * _OUT_W,), jnp.float32),
            pltpu.SemaphoreType.DMA,
            pltpu.SemaphoreType.DMA,
        ],
        compiler_params=pltpu.CompilerParams(needs_layout_passes=False),
    )
    flat = kern(xf, yw, tbl)
    return flat.reshape(_B, _OUT_W)


# speedup vs baseline: 1.7259x; 1.5853x over previous
import jax
import jax.numpy as jnp
from jax import lax
from jax.experimental import pallas as pl
from jax.experimental.pallas import tpu as pltpu
from jax.experimental.pallas import tpu_sc as plsc

_B = 16384
_XW = 16
_DIMS = (3, 4, 6, 4, 50)
_OUT_W = _XW + sum(_DIMS)

_NC, _NS = 2, 16
_NW = _NC * _NS
_BPW = _B // _NW
_NCHK = _BPW // 16

_TBASE = []
_a = 0
for _v, _d in zip((6, 7, 12, 7, 96), _DIMS):
    _TBASE.append(_a)
    _a += -(-(_v * _d) // 8) * 8
_TPACK = _a

_COL_OFF = []
_o = _XW
for _d in _DIMS:
    _COL_OFF.append(_o)
    _o += _d


def _body(xt_hbm, yw_hbm, tbl_hbm, out_hbm, tbl_v, y_v, stage_v, in_sem,
          st_sem):
    wid = lax.axis_index("s") * _NC + lax.axis_index("c")
    base = wid * _BPW

    c1 = pltpu.make_async_copy(tbl_hbm, tbl_v, in_sem)
    c2 = pltpu.make_async_copy(
        yw_hbm.at[pl.ds(wid * 5 * _BPW, 5 * _BPW)], y_v, in_sem)
    c3 = pltpu.make_async_copy(
        xt_hbm.at[:, pl.ds(base, _BPW)],
        stage_v.at[pl.ds(0, _XW), :], in_sem)
    c1.start()
    c2.start()
    c3.start()
    c1.wait()
    c2.wait()
    c3.wait()

    def chunk(c, carry):
        r0 = c * 16
        for t in range(5):
            d = _DIMS[t]
            yt = y_v[pl.ds(t * _BPW + r0, 16)]
            srcb = yt * d + _TBASE[t]
            for j in range(d):
                v = plsc.load_gather(tbl_v, [srcb + j])
                stage_v[_COL_OFF[t] + j, pl.ds(r0, 16)] = v
        return carry

    lax.fori_loop(0, _NCHK // 2, chunk, 0)
    s1 = pltpu.make_async_copy(
        stage_v.at[:, pl.ds(0, _BPW // 2)],
        out_hbm.at[:, pl.ds(base, _BPW // 2)], st_sem)
    s1.start()
    lax.fori_loop(_NCHK // 2, _NCHK, chunk, 0)
    s2 = pltpu.make_async_copy(
        stage_v.at[:, pl.ds(_BPW // 2, _BPW // 2)],
        out_hbm.at[:, pl.ds(base + _BPW // 2, _BPW // 2)], st_sem)
    s2.start()
    s1.wait()
    s2.wait()


def kernel(x, y, emb0, emb1, emb2, emb3, emb4):
    yw = y.T.reshape(5, _NW, _BPW).transpose(1, 0, 2).reshape(-1)
    xt = x.T
    pieces = []
    for e, b, nb in zip((emb0, emb1, emb2, emb3, emb4),
                        _TBASE, _TBASE[1:] + [_TPACK]):
        r = e.reshape(-1)
        pieces.append(r)
        pad = nb - b - r.shape[0]
        if pad:
            pieces.append(jnp.zeros((pad,), jnp.float32))
    tbl = jnp.concatenate(pieces)

    mesh = plsc.VectorSubcoreMesh(core_axis_name="c", subcore_axis_name="s")
    kern = pl.kernel(
        _body,
        out_type=jax.ShapeDtypeStruct((_OUT_W, _B), jnp.float32),
        mesh=mesh,
        scratch_types=[
            pltpu.VMEM((_TPACK,), jnp.float32),
            pltpu.VMEM((5 * _BPW,), jnp.int32),
            pltpu.VMEM((_OUT_W, _BPW), jnp.float32),
            pltpu.SemaphoreType.DMA,
            pltpu.SemaphoreType.DMA,
        ],
        compiler_params=pltpu.CompilerParams(
            needs_layout_passes=False, use_tc_tiling_on_sc=False),
    )
    out_t = kern(xt, yw, tbl)
    return out_t.T
